# Initial kernel scaffold; baseline (speedup 1.0000x reference)
import jax
import jax.numpy as jnp
from jax.experimental import pallas as pl


def kernel(x, gate_w, w1, w2):
    raise NotImplementedError("write your pallas kernel here")


# speedup vs baseline: 1.9484x; 1.9484x over previous
import functools

import jax
import jax.numpy as jnp
from jax import lax
from jax.experimental import pallas as pl
from jax.experimental.pallas import tpu as pltpu
from jax.experimental.pallas import tpu_sc as plsc

NUM_EXPERTS = 8
TOP_K = 2
AUX_COEF = 0.01

_NC = 2
_NS = 16
_NW = _NC * _NS
_L = 16

_TM = 256
_TI = 512


def _gelu_tanh(x):
    return 0.5 * x * (1.0 + jnp.tanh(jnp.sqrt(2.0 / jnp.pi) * (x + 0.044715 * x ** 3)))


def _router_body(x_ref, gw_ref, tw_ref, ti_ref, cnt_ref, psum_ref, aux_ref,
                 *, nblocks, n_tokens):
    i = pl.program_id(0)
    x = x_ref[...]
    logits = lax.dot_general(x, gw_ref[...], (((1,), (1,)), ((), ())),
                             preferred_element_type=jnp.float32)
    m = jnp.max(logits, axis=1, keepdims=True)
    ex = jnp.exp(logits - m)
    probs = ex / jnp.sum(ex, axis=1, keepdims=True)

    eids = lax.broadcasted_iota(jnp.int32, probs.shape, 1)
    m1 = jnp.max(probs, axis=1, keepdims=True)
    i1 = jnp.min(jnp.where(probs == m1, eids, NUM_EXPERTS), axis=1, keepdims=True)
    mask1 = eids == i1
    p2 = jnp.where(mask1, -1.0, probs)
    m2 = jnp.max(p2, axis=1, keepdims=True)
    i2 = jnp.min(jnp.where(p2 == m2, eids, NUM_EXPERTS), axis=1, keepdims=True)

    s = m1 + m2
    tw_ref[...] = jnp.concatenate([m1 / s, m2 / s], axis=1)
    ti_ref[...] = jnp.concatenate([i1, i2], axis=1).astype(jnp.int32)

    onehot = (mask1 | (eids == i2)).astype(jnp.float32)
    cnt = jnp.sum(onehot, axis=0, keepdims=True)
    ps = jnp.sum(probs, axis=0, keepdims=True)

    @pl.when(i == 0)
    def _():
        cnt_ref[...] = cnt
        psum_ref[...] = ps

    @pl.when(i > 0)
    def _():
        cnt_ref[...] = cnt_ref[...] + cnt
        psum_ref[...] = psum_ref[...] + ps

    @pl.when(i == nblocks - 1)
    def _():
        f = cnt_ref[...] / float(n_tokens * TOP_K)
        p = psum_ref[...] / float(n_tokens)
        aux_ref[...] = AUX_COEF * NUM_EXPERTS * jnp.sum(
            f * p, axis=1, keepdims=True)


def _router(xf, gate_w):
    n, h = xf.shape
    bt = 512
    nblocks = n // bt
    body = functools.partial(_router_body, nblocks=nblocks, n_tokens=n)
    return pl.pallas_call(
        body,
        grid=(nblocks,),
        in_specs=[
            pl.BlockSpec((bt, h), lambda i: (i, 0)),
            pl.BlockSpec((NUM_EXPERTS, h), lambda i: (0, 0)),
        ],
        out_specs=[
            pl.BlockSpec((bt, TOP_K), lambda i: (i, 0)),
            pl.BlockSpec((bt, TOP_K), lambda i: (i, 0)),
            pl.BlockSpec((1, NUM_EXPERTS), lambda i: (0, 0)),
            pl.BlockSpec((1, NUM_EXPERTS), lambda i: (0, 0)),
            pl.BlockSpec((1, 1), lambda i: (0, 0)),
        ],
        out_shape=[
            jax.ShapeDtypeStruct((n, TOP_K), jnp.float32),
            jax.ShapeDtypeStruct((n, TOP_K), jnp.int32),
            jax.ShapeDtypeStruct((1, NUM_EXPERTS), jnp.float32),
            jax.ShapeDtypeStruct((1, NUM_EXPERTS), jnp.float32),
            jax.ShapeDtypeStruct((1, 1), jnp.float32),
        ],
        compiler_params=pltpu.CompilerParams(
            dimension_semantics=("arbitrary",)),
    )(xf, gate_w)


def _sc_gather(table, idx):
    p, h = idx.shape[0], table.shape[1]
    per_w = p // _NW
    chunk = 64
    n_chunks = per_w // chunk
    mesh = plsc.VectorSubcoreMesh(core_axis_name="c", subcore_axis_name="s")

    @functools.partial(
        pl.kernel, mesh=mesh,
        out_type=jax.ShapeDtypeStruct((p, h), jnp.float32),
        scratch_types=[
            pltpu.VMEM((chunk,), jnp.int32),
            pltpu.VMEM((chunk, h), jnp.float32),
            pltpu.SemaphoreType.DMA,
        ],
    )
    def k(table_hbm, idx_hbm, out_hbm, idx_v, rows_v, sem):
        wid = lax.axis_index("s") * _NC + lax.axis_index("c")

        def body(ci, carry):
            base = wid * per_w + ci * chunk
            pltpu.sync_copy(idx_hbm.at[pl.ds(base, chunk)], idx_v)
            pltpu.async_copy(table_hbm.at[idx_v], rows_v, sem).wait()
            pltpu.sync_copy(rows_v, out_hbm.at[pl.ds(base, chunk)])
            return carry

        lax.fori_loop(0, n_chunks, body, 0)

    return k(table, idx)


def _mlp_body(s_ref, x_ref, w1_ref, w2_ref, wgt_ref, out_ref, *, nj):
    j = pl.program_id(1)
    x = x_ref[...]
    h = lax.dot_general(x, w1_ref[0], (((1,), (1,)), ((), ())),
                        preferred_element_type=jnp.float32)
    h = _gelu_tanh(h)
    contrib = lax.dot_general(h, w2_ref[0], (((1,), (1,)), ((), ())),
                              preferred_element_type=jnp.float32)

    @pl.when(j == 0)
    def _():
        out_ref[...] = contrib

    @pl.when(j > 0)
    def _():
        out_ref[...] = out_ref[...] + contrib

    @pl.when(j == nj - 1)
    def _():
        out_ref[...] = out_ref[...] * wgt_ref[...]


def _grouped_mlp(tile_expert, x_sorted, w1, w2, wsort):
    p, h = x_sorted.shape
    e, inter, _ = w1.shape
    nt = p // _TM
    nj = inter // _TI
    grid_spec = pltpu.PrefetchScalarGridSpec(
        num_scalar_prefetch=1,
        grid=(nt, nj),
        in_specs=[
            pl.BlockSpec((_TM, h), lambda i, j, s: (i, 0)),
            pl.BlockSpec((1, _TI, h), lambda i, j, s: (s[i], j, 0)),
            pl.BlockSpec((1, h, _TI), lambda i, j, s: (s[i], 0, j)),
            pl.BlockSpec((_TM, 1), lambda i, j, s: (i, 0)),
        ],
        out_specs=pl.BlockSpec((_TM, h), lambda i, j, s: (i, 0)),
    )
    return pl.pallas_call(
        functools.partial(_mlp_body, nj=nj),
        grid_spec=grid_spec,
        out_shape=jax.ShapeDtypeStruct((p, h), jnp.float32),
        compiler_params=pltpu.CompilerParams(
            dimension_semantics=("arbitrary", "arbitrary")),
    )(tile_expert, x_sorted, w1, w2, wsort)


def _sc_combine(rows, inv):
    n = inv.shape[0] // TOP_K
    h = rows.shape[1]
    per_w = n // _NW
    ct = 16
    n_chunks = per_w // ct
    mesh = plsc.VectorSubcoreMesh(core_axis_name="c", subcore_axis_name="s")

    @functools.partial(
        pl.kernel, mesh=mesh,
        out_type=jax.ShapeDtypeStruct((n, h), jnp.float32),
        scratch_types=[
            pltpu.VMEM((2 * ct,), jnp.int32),
            pltpu.VMEM((2 * ct, h), jnp.float32),
            pltpu.VMEM((ct, h), jnp.float32),
            pltpu.SemaphoreType.DMA,
        ],
    )
    def k(rows_hbm, inv_hbm, y_hbm, idx_v, rows_v, out_v, sem):
        wid = lax.axis_index("s") * _NC + lax.axis_index("c")

        def body(ci, carry):
            tbase = wid * per_w + ci * ct
            pltpu.sync_copy(inv_hbm.at[pl.ds(TOP_K * tbase, TOP_K * ct)], idx_v)
            pltpu.async_copy(rows_hbm.at[idx_v], rows_v, sem).wait()

            def tok(t, c2):
                def lane(cc, c3):
                    sl = pl.ds(cc * _L, _L)
                    out_v[t, sl] = rows_v[2 * t, sl] + rows_v[2 * t + 1, sl]
                    return c3
                return lax.fori_loop(0, h // _L, lane, c2)

            lax.fori_loop(0, ct, tok, 0)
            pltpu.sync_copy(out_v, y_hbm.at[pl.ds(tbase, ct)])
            return carry

        lax.fori_loop(0, n_chunks, body, 0)

    return k(rows, inv)


def kernel(x, gate_w, w1, w2):
    b, s, h = x.shape
    n = b * s
    xf = x.reshape(n, h)
    nk = n * TOP_K
    pad_n = nk + NUM_EXPERTS * _TM

    top_w, top_i, cnt, _psum, aux = _router(xf, gate_w)

    flat_idx = top_i.reshape(-1)
    sort_idx = jnp.argsort(flat_idx, stable=True)
    counts = cnt.reshape(-1).astype(jnp.int32)
    group_start = jnp.concatenate(
        [jnp.zeros((1,), jnp.int32), jnp.cumsum(counts)[:-1]])
    padded = ((counts + _TM - 1) // _TM) * _TM
    pad_off = jnp.concatenate(
        [jnp.zeros((1,), jnp.int32), jnp.cumsum(padded)[:-1]])
    e_sorted = flat_idx[sort_idx]
    ranks = jnp.arange(nk, dtype=jnp.int32) - group_start[e_sorted]
    pos = pad_off[e_sorted] + ranks
    src_tok = jnp.zeros((pad_n,), jnp.int32).at[pos].set(
        (sort_idx // TOP_K).astype(jnp.int32))
    wsort = jnp.zeros((pad_n,), jnp.float32).at[pos].set(
        top_w.reshape(-1)[sort_idx])
    inv = jnp.zeros((nk,), jnp.int32).at[sort_idx].set(pos)
    nt = pad_n // _TM
    tile_expert = (jnp.searchsorted(
        pad_off, jnp.arange(nt, dtype=jnp.int32) * _TM, side="right") - 1
    ).astype(jnp.int32)

    x_sorted = _sc_gather(xf, src_tok)
    y_sorted = _grouped_mlp(tile_expert, x_sorted, w1, w2,
                            wsort[:, None])
    y = _sc_combine(y_sorted, inv)

    return y.reshape(b, s, h), aux[0, 0]


# speedup vs baseline: 2.3981x; 1.2308x over previous
import functools

import jax
import jax.numpy as jnp
from jax import lax
from jax.experimental import pallas as pl
from jax.experimental.pallas import tpu as pltpu
from jax.experimental.pallas import tpu_sc as plsc

NUM_EXPERTS = 8
TOP_K = 2
AUX_COEF = 0.01

_NC = 2
_NS = 16
_NW = _NC * _NS
_L = 16

_TM = 512
_TI = 1024


def _gelu_tanh(x):
    return 0.5 * x * (1.0 + jnp.tanh(jnp.sqrt(2.0 / jnp.pi) * (x + 0.044715 * x ** 3)))


def _router_body(x_ref, gw_ref, tw_ref, ti_ref, rk_ref, cnt_ref, psum_ref,
                 aux_ref, *, nblocks, n_tokens):
    i = pl.program_id(0)
    x = x_ref[...]
    bt = x.shape[0]
    logits = lax.dot_general(x, gw_ref[...], (((1,), (1,)), ((), ())),
                             preferred_element_type=jnp.float32)
    m = jnp.max(logits, axis=1, keepdims=True)
    ex = jnp.exp(logits - m)
    probs = ex / jnp.sum(ex, axis=1, keepdims=True)

    eids = lax.broadcasted_iota(jnp.int32, probs.shape, 1)
    m1 = jnp.max(probs, axis=1, keepdims=True)
    i1 = jnp.min(jnp.where(probs == m1, eids, NUM_EXPERTS), axis=1, keepdims=True)
    mask1 = eids == i1
    p2 = jnp.where(mask1, -1.0, probs)
    m2 = jnp.max(p2, axis=1, keepdims=True)
    i2 = jnp.min(jnp.where(p2 == m2, eids, NUM_EXPERTS), axis=1, keepdims=True)

    s = m1 + m2
    tw_ref[...] = jnp.concatenate([m1 / s, m2 / s], axis=1)
    ti_ref[...] = jnp.concatenate([i1, i2], axis=1).astype(jnp.int32)

    oh1 = mask1.astype(jnp.float32)
    oh2 = (eids == i2).astype(jnp.float32)
    both = oh1 + oh2
    cnt = jnp.sum(both, axis=0, keepdims=True)
    ps = jnp.sum(probs, axis=0, keepdims=True)

    @pl.when(i == 0)
    def _():
        cnt_ref[...] = jnp.zeros_like(cnt)
        psum_ref[...] = jnp.zeros_like(ps)

    r_ids = lax.broadcasted_iota(jnp.int32, (bt, bt), 0)
    c_ids = lax.broadcasted_iota(jnp.int32, (bt, bt), 1)
    tril = (r_ids > c_ids).astype(jnp.float32)
    cums = lax.dot_general(tril, both, (((1,), (0,)), ((), ())),
                           preferred_element_type=jnp.float32)
    base = cums + cnt_ref[...]
    rank1 = jnp.sum(base * oh1, axis=1, keepdims=True)
    rank2 = jnp.sum(base * oh2, axis=1, keepdims=True)
    rk_ref[...] = jnp.concatenate([rank1, rank2], axis=1).astype(jnp.int32)

    cnt_ref[...] = cnt_ref[...] + cnt
    psum_ref[...] = psum_ref[...] + ps

    @pl.when(i == nblocks - 1)
    def _():
        f = cnt_ref[...] / float(n_tokens * TOP_K)
        p = psum_ref[...] / float(n_tokens)
        aux_ref[...] = AUX_COEF * NUM_EXPERTS * jnp.sum(
            f * p, axis=1, keepdims=True)


def _router(xf, gate_w):
    n, h = xf.shape
    bt = 512
    nblocks = n // bt
    body = functools.partial(_router_body, nblocks=nblocks, n_tokens=n)
    return pl.pallas_call(
        body,
        grid=(nblocks,),
        in_specs=[
            pl.BlockSpec((bt, h), lambda i: (i, 0)),
            pl.BlockSpec((NUM_EXPERTS, h), lambda i: (0, 0)),
        ],
        out_specs=[
            pl.BlockSpec((bt, TOP_K), lambda i: (i, 0)),
            pl.BlockSpec((bt, TOP_K), lambda i: (i, 0)),
            pl.BlockSpec((bt, TOP_K), lambda i: (i, 0)),
            pl.BlockSpec((1, NUM_EXPERTS), lambda i: (0, 0)),
            pl.BlockSpec((1, NUM_EXPERTS), lambda i: (0, 0)),
            pl.BlockSpec((1, 1), lambda i: (0, 0)),
        ],
        out_shape=[
            jax.ShapeDtypeStruct((n, TOP_K), jnp.float32),
            jax.ShapeDtypeStruct((n, TOP_K), jnp.int32),
            jax.ShapeDtypeStruct((n, TOP_K), jnp.int32),
            jax.ShapeDtypeStruct((1, NUM_EXPERTS), jnp.float32),
            jax.ShapeDtypeStruct((1, NUM_EXPERTS), jnp.float32),
            jax.ShapeDtypeStruct((1, 1), jnp.float32),
        ],
        compiler_params=pltpu.CompilerParams(
            dimension_semantics=("arbitrary",)),
    )(xf, gate_w)


def _sc_gather(table, idx):
    p, h = idx.shape[0], table.shape[1]
    per_w = p // _NW
    n_chunks = 8
    chunk = per_w // n_chunks
    mesh = plsc.VectorSubcoreMesh(core_axis_name="c", subcore_axis_name="s")

    @functools.partial(
        pl.kernel, mesh=mesh,
        out_type=jax.ShapeDtypeStruct((p, h), jnp.float32),
        scratch_types=[
            pltpu.VMEM((per_w,), jnp.int32),
            pltpu.VMEM((chunk, h), jnp.float32),
            pltpu.VMEM((chunk, h), jnp.float32),
            pltpu.SemaphoreType.DMA,
            pltpu.SemaphoreType.DMA,
        ],
    )
    def k(table_hbm, idx_hbm, out_hbm, idx_v, rows0, rows1, sem0, sem1):
        wid = lax.axis_index("s") * _NC + lax.axis_index("c")
        base = wid * per_w
        pltpu.sync_copy(idx_hbm.at[pl.ds(base, per_w)], idx_v)
        bufs = (rows0, rows1)
        sems = (sem0, sem1)
        handles = [None] * n_chunks
        handles[0] = pltpu.async_copy(
            table_hbm.at[idx_v.at[pl.ds(0, chunk)]], bufs[0], sems[0])
        for ci in range(n_chunks):
            if ci + 1 < n_chunks:
                handles[ci + 1] = pltpu.async_copy(
                    table_hbm.at[idx_v.at[pl.ds((ci + 1) * chunk, chunk)]],
                    bufs[(ci + 1) % 2], sems[(ci + 1) % 2])
            handles[ci].wait()
            pltpu.sync_copy(bufs[ci % 2],
                            out_hbm.at[pl.ds(base + ci * chunk, chunk)])

    return k(table, idx)


def _mlp_body(s_ref, x_ref, w1_ref, w2_ref, wgt_ref, out_ref, *, nj):
    j = pl.program_id(1)
    x = x_ref[...].astype(jnp.bfloat16)
    h = lax.dot_general(x, w1_ref[0].astype(jnp.bfloat16),
                        (((1,), (1,)), ((), ())),
                        preferred_element_type=jnp.float32)
    h = _gelu_tanh(h).astype(jnp.bfloat16)
    contrib = lax.dot_general(h, w2_ref[0].astype(jnp.bfloat16),
                              (((1,), (1,)), ((), ())),
                              preferred_element_type=jnp.float32)

    @pl.when(j == 0)
    def _():
        out_ref[...] = contrib

    @pl.when(j > 0)
    def _():
        out_ref[...] = out_ref[...] + contrib

    @pl.when(j == nj - 1)
    def _():
        out_ref[...] = out_ref[...] * wgt_ref[...]


def _grouped_mlp(tile_expert, x_sorted, w1, w2, wsort):
    p, h = x_sorted.shape
    e, inter, _ = w1.shape
    nt = p // _TM
    nj = inter // _TI
    grid_spec = pltpu.PrefetchScalarGridSpec(
        num_scalar_prefetch=1,
        grid=(nt, nj),
        in_specs=[
            pl.BlockSpec((_TM, h), lambda i, j, s: (i, 0)),
            pl.BlockSpec((1, _TI, h), lambda i, j, s: (s[i], j, 0)),
            pl.BlockSpec((1, h, _TI), lambda i, j, s: (s[i], 0, j)),
            pl.BlockSpec((_TM, 1), lambda i, j, s: (i, 0)),
        ],
        out_specs=pl.BlockSpec((_TM, h), lambda i, j, s: (i, 0)),
    )
    return pl.pallas_call(
        functools.partial(_mlp_body, nj=nj),
        grid_spec=grid_spec,
        out_shape=jax.ShapeDtypeStruct((p, h), jnp.float32),
        compiler_params=pltpu.CompilerParams(
            dimension_semantics=("arbitrary", "arbitrary")),
    )(tile_expert, x_sorted, w1, w2, wsort)


def _sc_combine(rows, inv):
    n = inv.shape[0] // TOP_K
    h = rows.shape[1]
    per_w = n // _NW
    ct = 16
    n_chunks = per_w // ct
    mesh = plsc.VectorSubcoreMesh(core_axis_name="c", subcore_axis_name="s")

    @functools.partial(
        pl.kernel, mesh=mesh,
        out_type=jax.ShapeDtypeStruct((n, h), jnp.float32),
        scratch_types=[
            pltpu.VMEM((TOP_K * per_w,), jnp.int32),
            pltpu.VMEM((TOP_K * ct, h), jnp.float32),
            pltpu.VMEM((TOP_K * ct, h), jnp.float32),
            pltpu.VMEM((ct, h), jnp.float32),
            pltpu.SemaphoreType.DMA,
            pltpu.SemaphoreType.DMA,
        ],
    )
    def k(rows_hbm, inv_hbm, y_hbm, idx_v, rv0, rv1, out_v, sem0, sem1):
        wid = lax.axis_index("s") * _NC + lax.axis_index("c")
        tbase = wid * per_w
        pltpu.sync_copy(inv_hbm.at[pl.ds(TOP_K * tbase, TOP_K * per_w)], idx_v)
        bufs = (rv0, rv1)
        sems = (sem0, sem1)
        handles = [None] * n_chunks
        handles[0] = pltpu.async_copy(
            rows_hbm.at[idx_v.at[pl.ds(0, TOP_K * ct)]], bufs[0], sems[0])
        for ci in range(n_chunks):
            if ci + 1 < n_chunks:
                handles[ci + 1] = pltpu.async_copy(
                    rows_hbm.at[idx_v.at[pl.ds((ci + 1) * TOP_K * ct,
                                               TOP_K * ct)]],
                    bufs[(ci + 1) % 2], sems[(ci + 1) % 2])
            handles[ci].wait()
            rows_v = bufs[ci % 2]

            def tok(t, c2):
                def lane(cc, c3):
                    sl = pl.ds(cc * _L, _L)
                    out_v[t, sl] = rows_v[2 * t, sl] + rows_v[2 * t + 1, sl]
                    return c3
                return lax.fori_loop(0, h // _L, lane, c2)

            lax.fori_loop(0, ct, tok, 0)
            pltpu.sync_copy(out_v, y_hbm.at[pl.ds(tbase + ci * ct, ct)])

    return k(rows, inv)


def kernel(x, gate_w, w1, w2):
    b, s, h = x.shape
    n = b * s
    xf = x.reshape(n, h)
    nk = n * TOP_K
    pad_n = nk + NUM_EXPERTS * _TM

    top_w, top_i, rank, cnt, _psum, aux = _router(xf, gate_w)

    flat_idx = top_i.reshape(-1)
    counts = cnt.reshape(-1).astype(jnp.int32)
    padded = ((counts + _TM - 1) // _TM) * _TM
    pad_off = jnp.concatenate(
        [jnp.zeros((1,), jnp.int32), jnp.cumsum(padded)[:-1]])
    pos = pad_off[flat_idx] + rank.reshape(-1)
    slot_tok = jnp.arange(nk, dtype=jnp.int32) // TOP_K
    src_tok = jnp.zeros((pad_n,), jnp.int32).at[pos].set(slot_tok)
    wsort = jnp.zeros((pad_n,), jnp.float32).at[pos].set(top_w.reshape(-1))
    inv = pos
    nt = pad_n // _TM
    tile_expert = (jnp.searchsorted(
        pad_off, jnp.arange(nt, dtype=jnp.int32) * _TM, side="right") - 1
    ).astype(jnp.int32)

    x_sorted = _sc_gather(xf, src_tok)
    y_sorted = _grouped_mlp(tile_expert, x_sorted, w1, w2,
                            wsort[:, None])
    y = _sc_combine(y_sorted, inv)

    return y.reshape(b, s, h), aux[0, 0]


# speedup vs baseline: 2.4062x; 1.0034x over previous
import functools

import jax
import jax.numpy as jnp
from jax import lax
from jax.experimental import pallas as pl
from jax.experimental.pallas import tpu as pltpu
from jax.experimental.pallas import tpu_sc as plsc

NUM_EXPERTS = 8
TOP_K = 2
AUX_COEF = 0.01

_NC = 2
_NS = 16
_NW = _NC * _NS
_L = 16

_TM = 512
_TI = 1024


def _gelu_tanh(x):
    return 0.5 * x * (1.0 + jnp.tanh(jnp.sqrt(2.0 / jnp.pi) * (x + 0.044715 * x ** 3)))


def _router_body(x_ref, gw_ref, tw_ref, ti_ref, rk_ref, cnt_ref, psum_ref,
                 aux_ref, *, nblocks, n_tokens):
    i = pl.program_id(0)
    x = x_ref[...]
    bt = x.shape[0]
    logits = lax.dot_general(x, gw_ref[...], (((1,), (1,)), ((), ())),
                             preferred_element_type=jnp.float32)
    m = jnp.max(logits, axis=1, keepdims=True)
    ex = jnp.exp(logits - m)
    probs = ex / jnp.sum(ex, axis=1, keepdims=True)

    eids = lax.broadcasted_iota(jnp.int32, probs.shape, 1)
    m1 = jnp.max(probs, axis=1, keepdims=True)
    i1 = jnp.min(jnp.where(probs == m1, eids, NUM_EXPERTS), axis=1, keepdims=True)
    mask1 = eids == i1
    p2 = jnp.where(mask1, -1.0, probs)
    m2 = jnp.max(p2, axis=1, keepdims=True)
    i2 = jnp.min(jnp.where(p2 == m2, eids, NUM_EXPERTS), axis=1, keepdims=True)

    s = m1 + m2
    tw_ref[...] = jnp.concatenate([m1 / s, m2 / s], axis=1)
    ti_ref[...] = jnp.concatenate([i1, i2], axis=1).astype(jnp.int32)

    oh1 = mask1.astype(jnp.float32)
    oh2 = (eids == i2).astype(jnp.float32)
    both = oh1 + oh2
    cnt = jnp.sum(both, axis=0, keepdims=True)
    ps = jnp.sum(probs, axis=0, keepdims=True)

    @pl.when(i == 0)
    def _():
        cnt_ref[...] = jnp.zeros_like(cnt)
        psum_ref[...] = jnp.zeros_like(ps)

    r_ids = lax.broadcasted_iota(jnp.int32, (bt, bt), 0)
    c_ids = lax.broadcasted_iota(jnp.int32, (bt, bt), 1)
    tril = (r_ids > c_ids).astype(jnp.bfloat16)
    cums = lax.dot_general(tril, both.astype(jnp.bfloat16),
                           (((1,), (0,)), ((), ())),
                           preferred_element_type=jnp.float32)
    base = cums + cnt_ref[...]
    rank1 = jnp.sum(base * oh1, axis=1, keepdims=True)
    rank2 = jnp.sum(base * oh2, axis=1, keepdims=True)
    rk_ref[...] = jnp.concatenate([rank1, rank2], axis=1).astype(jnp.int32)

    cnt_ref[...] = cnt_ref[...] + cnt
    psum_ref[...] = psum_ref[...] + ps

    @pl.when(i == nblocks - 1)
    def _():
        f = cnt_ref[...] / float(n_tokens * TOP_K)
        p = psum_ref[...] / float(n_tokens)
        aux_ref[...] = AUX_COEF * NUM_EXPERTS * jnp.sum(
            f * p, axis=1, keepdims=True)


def _router(xf, gate_w):
    n, h = xf.shape
    bt = 512
    nblocks = n // bt
    body = functools.partial(_router_body, nblocks=nblocks, n_tokens=n)
    return pl.pallas_call(
        body,
        grid=(nblocks,),
        in_specs=[
            pl.BlockSpec((bt, h), lambda i: (i, 0)),
            pl.BlockSpec((NUM_EXPERTS, h), lambda i: (0, 0)),
        ],
        out_specs=[
            pl.BlockSpec((bt, TOP_K), lambda i: (i, 0)),
            pl.BlockSpec((bt, TOP_K), lambda i: (i, 0)),
            pl.BlockSpec((bt, TOP_K), lambda i: (i, 0)),
            pl.BlockSpec((1, NUM_EXPERTS), lambda i: (0, 0)),
            pl.BlockSpec((1, NUM_EXPERTS), lambda i: (0, 0)),
            pl.BlockSpec((1, 1), lambda i: (0, 0)),
        ],
        out_shape=[
            jax.ShapeDtypeStruct((n, TOP_K), jnp.float32),
            jax.ShapeDtypeStruct((n, TOP_K), jnp.int32),
            jax.ShapeDtypeStruct((n, TOP_K), jnp.int32),
            jax.ShapeDtypeStruct((1, NUM_EXPERTS), jnp.float32),
            jax.ShapeDtypeStruct((1, NUM_EXPERTS), jnp.float32),
            jax.ShapeDtypeStruct((1, 1), jnp.float32),
        ],
        compiler_params=pltpu.CompilerParams(
            dimension_semantics=("arbitrary",)),
    )(xf, gate_w)


def _sc_gather(table, idx):
    p, h = idx.shape[0], table.shape[1]
    per_w = p // _NW
    n_chunks = 4
    chunk = per_w // n_chunks
    mesh = plsc.VectorSubcoreMesh(core_axis_name="c", subcore_axis_name="s")

    @functools.partial(
        pl.kernel, mesh=mesh,
        out_type=jax.ShapeDtypeStruct((p, h), jnp.float32),
        scratch_types=[
            pltpu.VMEM((per_w,), jnp.int32),
            pltpu.VMEM((chunk, h), jnp.float32),
            pltpu.SemaphoreType.DMA,
        ],
    )
    def k(table_hbm, idx_hbm, out_hbm, idx_v, rows_v, sem):
        wid = lax.axis_index("s") * _NC + lax.axis_index("c")
        base = wid * per_w
        pltpu.sync_copy(idx_hbm.at[pl.ds(base, per_w)], idx_v)
        for ci in range(n_chunks):
            pltpu.async_copy(
                table_hbm.at[idx_v.at[pl.ds(ci * chunk, chunk)]],
                rows_v, sem).wait()
            pltpu.sync_copy(rows_v,
                            out_hbm.at[pl.ds(base + ci * chunk, chunk)])

    return k(table, idx)


def _mlp_body(s_ref, x_ref, w1_ref, w2_ref, wgt_ref, out_ref, *, nj):
    j = pl.program_id(1)
    x = x_ref[...].astype(jnp.bfloat16)
    h = lax.dot_general(x, w1_ref[0].astype(jnp.bfloat16),
                        (((1,), (1,)), ((), ())),
                        preferred_element_type=jnp.float32)
    h = _gelu_tanh(h).astype(jnp.bfloat16)
    contrib = lax.dot_general(h, w2_ref[0].astype(jnp.bfloat16),
                              (((1,), (1,)), ((), ())),
                              preferred_element_type=jnp.float32)

    @pl.when(j == 0)
    def _():
        out_ref[...] = contrib

    @pl.when(j > 0)
    def _():
        out_ref[...] = out_ref[...] + contrib

    @pl.when(j == nj - 1)
    def _():
        out_ref[...] = out_ref[...] * wgt_ref[...]


def _grouped_mlp(tile_expert, x_sorted, w1, w2, wsort):
    p, h = x_sorted.shape
    e, inter, _ = w1.shape
    nt = p // _TM
    nj = inter // _TI
    grid_spec = pltpu.PrefetchScalarGridSpec(
        num_scalar_prefetch=1,
        grid=(nt, nj),
        in_specs=[
            pl.BlockSpec((_TM, h), lambda i, j, s: (i, 0)),
            pl.BlockSpec((1, _TI, h), lambda i, j, s: (s[i], j, 0)),
            pl.BlockSpec((1, h, _TI), lambda i, j, s: (s[i], 0, j)),
            pl.BlockSpec((_TM, 1), lambda i, j, s: (i, 0)),
        ],
        out_specs=pl.BlockSpec((_TM, h), lambda i, j, s: (i, 0)),
    )
    return pl.pallas_call(
        functools.partial(_mlp_body, nj=nj),
        grid_spec=grid_spec,
        out_shape=jax.ShapeDtypeStruct((p, h), jnp.float32),
        compiler_params=pltpu.CompilerParams(
            dimension_semantics=("arbitrary", "arbitrary")),
    )(tile_expert, x_sorted, w1, w2, wsort)


def _sc_combine(rows, inv):
    n = inv.shape[0] // TOP_K
    h = rows.shape[1]
    per_w = n // _NW
    ct = 16
    n_chunks = per_w // ct
    mesh = plsc.VectorSubcoreMesh(core_axis_name="c", subcore_axis_name="s")

    @functools.partial(
        pl.kernel, mesh=mesh,
        out_type=jax.ShapeDtypeStruct((n, h), jnp.float32),
        scratch_types=[
            pltpu.VMEM((TOP_K * per_w,), jnp.int32),
            pltpu.VMEM((TOP_K * ct, h), jnp.float32),
            pltpu.VMEM((TOP_K * ct, h), jnp.float32),
            pltpu.VMEM((ct, h), jnp.float32),
            pltpu.SemaphoreType.DMA,
            pltpu.SemaphoreType.DMA,
        ],
    )
    def k(rows_hbm, inv_hbm, y_hbm, idx_v, rv0, rv1, out_v, sem0, sem1):
        wid = lax.axis_index("s") * _NC + lax.axis_index("c")
        tbase = wid * per_w
        pltpu.sync_copy(inv_hbm.at[pl.ds(TOP_K * tbase, TOP_K * per_w)], idx_v)
        bufs = (rv0, rv1)
        sems = (sem0, sem1)
        handles = [None] * n_chunks
        handles[0] = pltpu.async_copy(
            rows_hbm.at[idx_v.at[pl.ds(0, TOP_K * ct)]], bufs[0], sems[0])
        for ci in range(n_chunks):
            if ci + 1 < n_chunks:
                handles[ci + 1] = pltpu.async_copy(
                    rows_hbm.at[idx_v.at[pl.ds((ci + 1) * TOP_K * ct,
                                               TOP_K * ct)]],
                    bufs[(ci + 1) % 2], sems[(ci + 1) % 2])
            handles[ci].wait()
            rows_v = bufs[ci % 2]

            def tok(t, c2):
                def lane(cc, c3):
                    for u in range(4):
                        sl = pl.ds((cc * 4 + u) * _L, _L)
                        out_v[t, sl] = rows_v[2 * t, sl] + rows_v[2 * t + 1, sl]
                    return c3
                return lax.fori_loop(0, h // (4 * _L), lane, c2)

            lax.fori_loop(0, ct, tok, 0)
            pltpu.sync_copy(out_v, y_hbm.at[pl.ds(tbase + ci * ct, ct)])

    return k(rows, inv)


def kernel(x, gate_w, w1, w2):
    b, s, h = x.shape
    n = b * s
    xf = x.reshape(n, h)
    nk = n * TOP_K
    pad_n = nk + NUM_EXPERTS * _TM

    top_w, top_i, rank, cnt, _psum, aux = _router(xf, gate_w)

    flat_idx = top_i.reshape(-1)
    counts = cnt.reshape(-1).astype(jnp.int32)
    padded = ((counts + _TM - 1) // _TM) * _TM
    pad_off = jnp.concatenate(
        [jnp.zeros((1,), jnp.int32), jnp.cumsum(padded)[:-1]])
    pos = pad_off[flat_idx] + rank.reshape(-1)
    slot_tok = jnp.arange(nk, dtype=jnp.int32) // TOP_K
    src_tok = jnp.zeros((pad_n,), jnp.int32).at[pos].set(slot_tok)
    wsort = jnp.zeros((pad_n,), jnp.float32).at[pos].set(top_w.reshape(-1))
    inv = pos
    nt = pad_n // _TM
    tile_expert = (jnp.searchsorted(
        pad_off, jnp.arange(nt, dtype=jnp.int32) * _TM, side="right") - 1
    ).astype(jnp.int32)

    x_sorted = _sc_gather(xf, src_tok)
    y_sorted = _grouped_mlp(tile_expert, x_sorted, w1, w2,
                            wsort[:, None])
    y = _sc_combine(y_sorted, inv)

    return y.reshape(b, s, h), aux[0, 0]


# speedup vs baseline: 3.3774x; 1.4036x over previous
import functools

import jax
import jax.numpy as jnp
from jax import lax
from jax.experimental import pallas as pl
from jax.experimental.pallas import tpu as pltpu
from jax.experimental.pallas import tpu_sc as plsc

NUM_EXPERTS = 8
TOP_K = 2
AUX_COEF = 0.01

_NC = 2
_NS = 16
_NW = _NC * _NS
_L = 16

_TM = 512
_TI = 1024


def _gelu_tanh(x):
    return 0.5 * x * (1.0 + jnp.tanh(jnp.sqrt(2.0 / jnp.pi) * (x + 0.044715 * x ** 3)))


def _router_body(x_ref, gw_ref, tw_ref, ti_ref, rk_ref, cnt_ref, psum_ref,
                 aux_ref, *, nblocks, n_tokens):
    i = pl.program_id(0)
    x = x_ref[...]
    bt = x.shape[0]
    logits = lax.dot_general(x, gw_ref[...], (((1,), (1,)), ((), ())),
                             preferred_element_type=jnp.float32)
    m = jnp.max(logits, axis=1, keepdims=True)
    ex = jnp.exp(logits - m)
    probs = ex / jnp.sum(ex, axis=1, keepdims=True)

    eids = lax.broadcasted_iota(jnp.int32, probs.shape, 1)
    m1 = jnp.max(probs, axis=1, keepdims=True)
    i1 = jnp.min(jnp.where(probs == m1, eids, NUM_EXPERTS), axis=1, keepdims=True)
    mask1 = eids == i1
    p2 = jnp.where(mask1, -1.0, probs)
    m2 = jnp.max(p2, axis=1, keepdims=True)
    i2 = jnp.min(jnp.where(p2 == m2, eids, NUM_EXPERTS), axis=1, keepdims=True)

    s = m1 + m2
    tw_ref[...] = jnp.concatenate([m1 / s, m2 / s], axis=1)
    ti_ref[...] = jnp.concatenate([i1, i2], axis=1).astype(jnp.int32)

    oh1 = mask1.astype(jnp.float32)
    oh2 = (eids == i2).astype(jnp.float32)
    both = oh1 + oh2
    cnt = jnp.sum(both, axis=0, keepdims=True)
    ps = jnp.sum(probs, axis=0, keepdims=True)

    @pl.when(i == 0)
    def _():
        cnt_ref[...] = jnp.zeros_like(cnt)
        psum_ref[...] = jnp.zeros_like(ps)

    r_ids = lax.broadcasted_iota(jnp.int32, (bt, bt), 0)
    c_ids = lax.broadcasted_iota(jnp.int32, (bt, bt), 1)
    tril = (r_ids > c_ids).astype(jnp.bfloat16)
    cums = lax.dot_general(tril, both.astype(jnp.bfloat16),
                           (((1,), (0,)), ((), ())),
                           preferred_element_type=jnp.float32)
    base = cums + cnt_ref[...]
    rank1 = jnp.sum(base * oh1, axis=1, keepdims=True)
    rank2 = jnp.sum(base * oh2, axis=1, keepdims=True)
    rk_ref[...] = jnp.concatenate([rank1, rank2], axis=1).astype(jnp.int32)

    cnt_ref[...] = cnt_ref[...] + cnt
    psum_ref[...] = psum_ref[...] + ps

    @pl.when(i == nblocks - 1)
    def _():
        f = cnt_ref[...] / float(n_tokens * TOP_K)
        p = psum_ref[...] / float(n_tokens)
        aux_ref[...] = AUX_COEF * NUM_EXPERTS * jnp.sum(
            f * p, axis=1, keepdims=True)


def _router(xf, gate_w):
    n, h = xf.shape
    bt = 512
    nblocks = n // bt
    body = functools.partial(_router_body, nblocks=nblocks, n_tokens=n)
    return pl.pallas_call(
        body,
        grid=(nblocks,),
        in_specs=[
            pl.BlockSpec((bt, h), lambda i: (i, 0)),
            pl.BlockSpec((NUM_EXPERTS, h), lambda i: (0, 0)),
        ],
        out_specs=[
            pl.BlockSpec((bt, TOP_K), lambda i: (i, 0)),
            pl.BlockSpec((bt, TOP_K), lambda i: (i, 0)),
            pl.BlockSpec((bt, TOP_K), lambda i: (i, 0)),
            pl.BlockSpec((1, NUM_EXPERTS), lambda i: (0, 0)),
            pl.BlockSpec((1, NUM_EXPERTS), lambda i: (0, 0)),
            pl.BlockSpec((1, 1), lambda i: (0, 0)),
        ],
        out_shape=[
            jax.ShapeDtypeStruct((n, TOP_K), jnp.float32),
            jax.ShapeDtypeStruct((n, TOP_K), jnp.int32),
            jax.ShapeDtypeStruct((n, TOP_K), jnp.int32),
            jax.ShapeDtypeStruct((1, NUM_EXPERTS), jnp.float32),
            jax.ShapeDtypeStruct((1, NUM_EXPERTS), jnp.float32),
            jax.ShapeDtypeStruct((1, 1), jnp.float32),
        ],
        compiler_params=pltpu.CompilerParams(
            dimension_semantics=("arbitrary",)),
    )(xf, gate_w)


def _sc_gather(table, idx):
    p, h = idx.shape[0], table.shape[1]
    per_w = p // _NW
    n_chunks = 4
    chunk = per_w // n_chunks
    mesh = plsc.VectorSubcoreMesh(core_axis_name="c", subcore_axis_name="s")

    @functools.partial(
        pl.kernel, mesh=mesh,
        out_type=jax.ShapeDtypeStruct((p, h), jnp.float32),
        scratch_types=[
            pltpu.VMEM((chunk,), jnp.int32),
            pltpu.VMEM((chunk, h), jnp.float32),
            pltpu.SemaphoreType.DMA,
        ],
    )
    def k(table_hbm, idx_hbm, out_hbm, idx_v, rows_v, sem):
        wid = lax.axis_index("s") * _NC + lax.axis_index("c")
        base = wid * per_w

        def body(ci, carry):
            pltpu.sync_copy(idx_hbm.at[pl.ds(base + ci * chunk, chunk)], idx_v)
            pltpu.async_copy(table_hbm.at[idx_v], rows_v, sem).wait()
            pltpu.sync_copy(rows_v,
                            out_hbm.at[pl.ds(base + ci * chunk, chunk)])
            return carry

        lax.fori_loop(0, n_chunks, body, 0)

    return k(table, idx)


def _mlp_body(s_ref, x_ref, w1_ref, w2_ref, wgt_ref, out_ref, *, nj):
    j = pl.program_id(1)
    x = x_ref[...].astype(jnp.bfloat16)
    h = lax.dot_general(x, w1_ref[0].astype(jnp.bfloat16),
                        (((1,), (1,)), ((), ())),
                        preferred_element_type=jnp.float32)
    h = _gelu_tanh(h.astype(jnp.bfloat16))
    contrib = lax.dot_general(h, w2_ref[0].astype(jnp.bfloat16),
                              (((1,), (1,)), ((), ())),
                              preferred_element_type=jnp.float32)

    @pl.when(j == 0)
    def _():
        out_ref[...] = contrib

    @pl.when(j > 0)
    def _():
        out_ref[...] = out_ref[...] + contrib

    @pl.when(j == nj - 1)
    def _():
        out_ref[...] = out_ref[...] * wgt_ref[...]


def _grouped_mlp(tile_expert, x_sorted, w1, w2, wsort):
    p, h = x_sorted.shape
    e, inter, _ = w1.shape
    nt = p // _TM
    nj = inter // _TI
    grid_spec = pltpu.PrefetchScalarGridSpec(
        num_scalar_prefetch=1,
        grid=(nt, nj),
        in_specs=[
            pl.BlockSpec((_TM, h), lambda i, j, s: (i, 0)),
            pl.BlockSpec((1, _TI, h), lambda i, j, s: (s[i], j, 0)),
            pl.BlockSpec((1, h, _TI), lambda i, j, s: (s[i], 0, j)),
            pl.BlockSpec((_TM, 1), lambda i, j, s: (i, 0)),
        ],
        out_specs=pl.BlockSpec((_TM, h), lambda i, j, s: (i, 0)),
    )
    return pl.pallas_call(
        functools.partial(_mlp_body, nj=nj),
        grid_spec=grid_spec,
        out_shape=jax.ShapeDtypeStruct((p, h), jnp.float32),
        compiler_params=pltpu.CompilerParams(
            dimension_semantics=("arbitrary", "arbitrary")),
    )(tile_expert, x_sorted, w1, w2, wsort)


def _sc_combine(rows, inv):
    n = inv.shape[0] // TOP_K
    h = rows.shape[1]
    per_w = n // _NW
    ct = 16
    n_chunks = per_w // ct
    mesh = plsc.VectorSubcoreMesh(core_axis_name="c", subcore_axis_name="s")

    @functools.partial(
        pl.kernel, mesh=mesh,
        out_type=jax.ShapeDtypeStruct((n, h), jnp.float32),
        scratch_types=[
            pltpu.VMEM((TOP_K * per_w,), jnp.int32),
            pltpu.VMEM((TOP_K * ct, h), jnp.float32),
            pltpu.VMEM((TOP_K * ct, h), jnp.float32),
            pltpu.VMEM((ct, h), jnp.float32),
            pltpu.SemaphoreType.DMA,
            pltpu.SemaphoreType.DMA,
        ],
    )
    def k(rows_hbm, inv_hbm, y_hbm, idx_v, rv0, rv1, out_v, sem0, sem1):
        wid = lax.axis_index("s") * _NC + lax.axis_index("c")
        tbase = wid * per_w
        pltpu.sync_copy(inv_hbm.at[pl.ds(TOP_K * tbase, TOP_K * per_w)], idx_v)
        bufs = (rv0, rv1)
        sems = (sem0, sem1)
        handles = [None] * n_chunks
        handles[0] = pltpu.async_copy(
            rows_hbm.at[idx_v.at[pl.ds(0, TOP_K * ct)]], bufs[0], sems[0])
        for ci in range(n_chunks):
            if ci + 1 < n_chunks:
                handles[ci + 1] = pltpu.async_copy(
                    rows_hbm.at[idx_v.at[pl.ds((ci + 1) * TOP_K * ct,
                                               TOP_K * ct)]],
                    bufs[(ci + 1) % 2], sems[(ci + 1) % 2])
            handles[ci].wait()
            rows_v = bufs[ci % 2]

            def tok(t, c2):
                def lane(cc, c3):
                    for u in range(4):
                        sl = pl.ds((cc * 4 + u) * _L, _L)
                        out_v[t, sl] = rows_v[2 * t, sl] + rows_v[2 * t + 1, sl]
                    return c3
                return lax.fori_loop(0, h // (4 * _L), lane, c2)

            lax.fori_loop(0, ct, tok, 0)
            pltpu.sync_copy(out_v, y_hbm.at[pl.ds(tbase + ci * ct, ct)])

    return k(rows, inv)


def kernel(x, gate_w, w1, w2):
    b, s, h = x.shape
    n = b * s
    xf = x.reshape(n, h)
    nk = n * TOP_K
    pad_n = nk + NUM_EXPERTS * _TM

    top_w, top_i, rank, cnt, _psum, aux = _router(xf, gate_w)

    flat_idx = top_i.reshape(-1)
    counts = cnt.reshape(-1).astype(jnp.int32)
    padded = ((counts + _TM - 1) // _TM) * _TM
    pad_off = jnp.concatenate(
        [jnp.zeros((1,), jnp.int32), jnp.cumsum(padded)[:-1]])
    pos = pad_off[flat_idx] + rank.reshape(-1)
    slot_tok = jnp.arange(nk, dtype=jnp.int32) // TOP_K
    src_tok = (jnp.arange(pad_n, dtype=jnp.int32) % n).at[pos].set(slot_tok)
    wsort = jnp.zeros((pad_n,), jnp.float32).at[pos].set(top_w.reshape(-1))
    inv = pos
    nt = pad_n // _TM
    tile_expert = (jnp.searchsorted(
        pad_off, jnp.arange(nt, dtype=jnp.int32) * _TM, side="right") - 1
    ).astype(jnp.int32)

    x_sorted = _sc_gather(xf, src_tok)
    y_sorted = _grouped_mlp(tile_expert, x_sorted, w1, w2,
                            wsort[:, None])
    y = _sc_combine(y_sorted, inv)

    return y.reshape(b, s, h), aux[0, 0]


# speedup vs baseline: 3.6308x; 1.0750x over previous
import functools

import jax
import jax.numpy as jnp
from jax import lax
from jax.experimental import pallas as pl
from jax.experimental.pallas import tpu as pltpu
from jax.experimental.pallas import tpu_sc as plsc

NUM_EXPERTS = 8
TOP_K = 2
AUX_COEF = 0.01

_NC = 2
_NS = 16
_NW = _NC * _NS
_L = 16

_TM = 512
_TI = 2048


def _gelu_tanh(x):
    return 0.5 * x * (1.0 + jnp.tanh(jnp.sqrt(2.0 / jnp.pi) * (x + 0.044715 * x ** 3)))


def _router_body(x_ref, gw_ref, tw_ref, ti_ref, rk_ref, cnt_ref, psum_ref,
                 aux_ref, *, nblocks, n_tokens):
    i = pl.program_id(0)
    x = x_ref[...]
    bt = x.shape[0]
    logits = lax.dot_general(x, gw_ref[...], (((1,), (1,)), ((), ())),
                             preferred_element_type=jnp.float32)
    m = jnp.max(logits, axis=1, keepdims=True)
    ex = jnp.exp(logits - m)
    probs = ex / jnp.sum(ex, axis=1, keepdims=True)

    eids = lax.broadcasted_iota(jnp.int32, probs.shape, 1)
    m1 = jnp.max(probs, axis=1, keepdims=True)
    i1 = jnp.min(jnp.where(probs == m1, eids, NUM_EXPERTS), axis=1, keepdims=True)
    mask1 = eids == i1
    p2 = jnp.where(mask1, -1.0, probs)
    m2 = jnp.max(p2, axis=1, keepdims=True)
    i2 = jnp.min(jnp.where(p2 == m2, eids, NUM_EXPERTS), axis=1, keepdims=True)

    s = m1 + m2
    tw_ref[...] = jnp.concatenate([m1 / s, m2 / s], axis=1)
    ti_ref[...] = jnp.concatenate([i1, i2], axis=1).astype(jnp.int32)

    oh1 = mask1.astype(jnp.float32)
    oh2 = (eids == i2).astype(jnp.float32)
    both = oh1 + oh2
    cnt = jnp.sum(both, axis=0, keepdims=True)
    ps = jnp.sum(probs, axis=0, keepdims=True)

    @pl.when(i == 0)
    def _():
        cnt_ref[...] = jnp.zeros_like(cnt)
        psum_ref[...] = jnp.zeros_like(ps)

    r_ids = lax.broadcasted_iota(jnp.int32, (bt, bt), 0)
    c_ids = lax.broadcasted_iota(jnp.int32, (bt, bt), 1)
    tril = (r_ids > c_ids).astype(jnp.bfloat16)
    cums = lax.dot_general(tril, both.astype(jnp.bfloat16),
                           (((1,), (0,)), ((), ())),
                           preferred_element_type=jnp.float32)
    base = cums + cnt_ref[...]
    rank1 = jnp.sum(base * oh1, axis=1, keepdims=True)
    rank2 = jnp.sum(base * oh2, axis=1, keepdims=True)
    rk_ref[...] = jnp.concatenate([rank1, rank2], axis=1).astype(jnp.int32)

    cnt_ref[...] = cnt_ref[...] + cnt
    psum_ref[...] = psum_ref[...] + ps

    @pl.when(i == nblocks - 1)
    def _():
        f = cnt_ref[...] / float(n_tokens * TOP_K)
        p = psum_ref[...] / float(n_tokens)
        aux_ref[...] = AUX_COEF * NUM_EXPERTS * jnp.sum(
            f * p, axis=1, keepdims=True)


def _router(xf, gate_w):
    n, h = xf.shape
    bt = 512
    nblocks = n // bt
    body = functools.partial(_router_body, nblocks=nblocks, n_tokens=n)
    return pl.pallas_call(
        body,
        grid=(nblocks,),
        in_specs=[
            pl.BlockSpec((bt, h), lambda i: (i, 0)),
            pl.BlockSpec((NUM_EXPERTS, h), lambda i: (0, 0)),
        ],
        out_specs=[
            pl.BlockSpec((bt, TOP_K), lambda i: (i, 0)),
            pl.BlockSpec((bt, TOP_K), lambda i: (i, 0)),
            pl.BlockSpec((bt, TOP_K), lambda i: (i, 0)),
            pl.BlockSpec((1, NUM_EXPERTS), lambda i: (0, 0)),
            pl.BlockSpec((1, NUM_EXPERTS), lambda i: (0, 0)),
            pl.BlockSpec((1, 1), lambda i: (0, 0)),
        ],
        out_shape=[
            jax.ShapeDtypeStruct((n, TOP_K), jnp.float32),
            jax.ShapeDtypeStruct((n, TOP_K), jnp.int32),
            jax.ShapeDtypeStruct((n, TOP_K), jnp.int32),
            jax.ShapeDtypeStruct((1, NUM_EXPERTS), jnp.float32),
            jax.ShapeDtypeStruct((1, NUM_EXPERTS), jnp.float32),
            jax.ShapeDtypeStruct((1, 1), jnp.float32),
        ],
        compiler_params=pltpu.CompilerParams(
            dimension_semantics=("arbitrary",)),
    )(xf, gate_w)


def _sc_gather(table, idx):
    p, h = idx.shape[0], table.shape[1]
    per_w = p // _NW
    n_chunks = 4
    chunk = per_w // n_chunks
    mesh = plsc.VectorSubcoreMesh(core_axis_name="c", subcore_axis_name="s")

    @functools.partial(
        pl.kernel, mesh=mesh,
        out_type=jax.ShapeDtypeStruct((p, h), jnp.float32),
        scratch_types=[
            pltpu.VMEM((chunk,), jnp.int32),
            pltpu.VMEM((chunk, h), jnp.float32),
            pltpu.SemaphoreType.DMA,
        ],
    )
    def k(table_hbm, idx_hbm, out_hbm, idx_v, rows_v, sem):
        wid = lax.axis_index("s") * _NC + lax.axis_index("c")
        base = wid * per_w

        def body(ci, carry):
            pltpu.sync_copy(idx_hbm.at[pl.ds(base + ci * chunk, chunk)], idx_v)
            pltpu.async_copy(table_hbm.at[idx_v], rows_v, sem).wait()
            pltpu.sync_copy(rows_v,
                            out_hbm.at[pl.ds(base + ci * chunk, chunk)])
            return carry

        lax.fori_loop(0, n_chunks, body, 0)

    return k(table, idx)


def _mlp_body(s_ref, x_ref, w1_ref, w2_ref, wgt_ref, out_ref, *, nj):
    j = pl.program_id(1)
    x = x_ref[...].astype(jnp.bfloat16)
    h = lax.dot_general(x, w1_ref[0].astype(jnp.bfloat16),
                        (((1,), (1,)), ((), ())),
                        preferred_element_type=jnp.float32)
    h = _gelu_tanh(h.astype(jnp.bfloat16))
    contrib = lax.dot_general(h, w2_ref[0].astype(jnp.bfloat16),
                              (((1,), (1,)), ((), ())),
                              preferred_element_type=jnp.float32)

    @pl.when(j == 0)
    def _():
        out_ref[...] = contrib

    @pl.when(j > 0)
    def _():
        out_ref[...] = out_ref[...] + contrib

    @pl.when(j == nj - 1)
    def _():
        out_ref[...] = out_ref[...] * wgt_ref[...]


def _grouped_mlp(tile_expert, x_sorted, w1, w2, wsort):
    p, h = x_sorted.shape
    e, inter, _ = w1.shape
    nt = p // _TM
    nj = inter // _TI
    grid_spec = pltpu.PrefetchScalarGridSpec(
        num_scalar_prefetch=1,
        grid=(nt, nj),
        in_specs=[
            pl.BlockSpec((_TM, h), lambda i, j, s: (i, 0)),
            pl.BlockSpec((1, _TI, h), lambda i, j, s: (s[i], j, 0)),
            pl.BlockSpec((1, h, _TI), lambda i, j, s: (s[i], 0, j)),
            pl.BlockSpec((_TM, 1), lambda i, j, s: (i, 0)),
        ],
        out_specs=pl.BlockSpec((_TM, h), lambda i, j, s: (i, 0)),
    )
    return pl.pallas_call(
        functools.partial(_mlp_body, nj=nj),
        grid_spec=grid_spec,
        out_shape=jax.ShapeDtypeStruct((p, h), jnp.float32),
        compiler_params=pltpu.CompilerParams(
            dimension_semantics=("arbitrary", "arbitrary")),
    )(tile_expert, x_sorted, w1, w2, wsort)


def _sc_combine(rows, inv):
    n = inv.shape[0] // TOP_K
    h = rows.shape[1]
    per_w = n // _NW
    ct = 16
    n_chunks = per_w // ct
    mesh = plsc.VectorSubcoreMesh(core_axis_name="c", subcore_axis_name="s")

    @functools.partial(
        pl.kernel, mesh=mesh,
        out_type=jax.ShapeDtypeStruct((n, h), jnp.float32),
        scratch_types=[
            pltpu.VMEM((TOP_K * per_w,), jnp.int32),
            pltpu.VMEM((TOP_K * ct, h), jnp.float32),
            pltpu.VMEM((TOP_K * ct, h), jnp.float32),
            pltpu.VMEM((ct, h), jnp.float32),
            pltpu.SemaphoreType.DMA,
            pltpu.SemaphoreType.DMA,
        ],
    )
    def k(rows_hbm, inv_hbm, y_hbm, idx_v, rv0, rv1, out_v, sem0, sem1):
        wid = lax.axis_index("s") * _NC + lax.axis_index("c")
        tbase = wid * per_w
        pltpu.sync_copy(inv_hbm.at[pl.ds(TOP_K * tbase, TOP_K * per_w)], idx_v)
        bufs = (rv0, rv1)
        sems = (sem0, sem1)
        handles = [None] * n_chunks
        handles[0] = pltpu.async_copy(
            rows_hbm.at[idx_v.at[pl.ds(0, TOP_K * ct)]], bufs[0], sems[0])
        for ci in range(n_chunks):
            if ci + 1 < n_chunks:
                handles[ci + 1] = pltpu.async_copy(
                    rows_hbm.at[idx_v.at[pl.ds((ci + 1) * TOP_K * ct,
                                               TOP_K * ct)]],
                    bufs[(ci + 1) % 2], sems[(ci + 1) % 2])
            handles[ci].wait()
            rows_v = bufs[ci % 2]

            def tok(t, c2):
                def lane(cc, c3):
                    for u in range(4):
                        sl = pl.ds((cc * 4 + u) * _L, _L)
                        out_v[t, sl] = rows_v[2 * t, sl] + rows_v[2 * t + 1, sl]
                    return c3
                return lax.fori_loop(0, h // (4 * _L), lane, c2)

            lax.fori_loop(0, ct, tok, 0)
            pltpu.sync_copy(out_v, y_hbm.at[pl.ds(tbase + ci * ct, ct)])

    return k(rows, inv)


def kernel(x, gate_w, w1, w2):
    b, s, h = x.shape
    n = b * s
    xf = x.reshape(n, h)
    nk = n * TOP_K
    pad_n = nk + NUM_EXPERTS * _TM

    top_w, top_i, rank, cnt, _psum, aux = _router(xf, gate_w)

    flat_idx = top_i.reshape(-1)
    counts = cnt.reshape(-1).astype(jnp.int32)
    padded = ((counts + _TM - 1) // _TM) * _TM
    pad_off = jnp.concatenate(
        [jnp.zeros((1,), jnp.int32), jnp.cumsum(padded)[:-1]])
    pos = pad_off[flat_idx] + rank.reshape(-1)
    slot_tok = jnp.arange(nk, dtype=jnp.int32) // TOP_K
    src_tok = (jnp.arange(pad_n, dtype=jnp.int32) % n).at[pos].set(slot_tok)
    wsort = jnp.zeros((pad_n,), jnp.float32).at[pos].set(top_w.reshape(-1))
    inv = pos
    nt = pad_n // _TM
    tile_expert = (jnp.searchsorted(
        pad_off, jnp.arange(nt, dtype=jnp.int32) * _TM, side="right") - 1
    ).astype(jnp.int32)

    x_sorted = _sc_gather(xf, src_tok)
    y_sorted = _grouped_mlp(tile_expert, x_sorted, w1, w2,
                            wsort[:, None])
    y = _sc_combine(y_sorted, inv)

    return y.reshape(b, s, h), aux[0, 0]
